# baseline (speedup 1.0000x reference)
import functools

import jax
import jax.numpy as jnp
import numpy as np
from jax import lax
from jax.experimental import pallas as pl
from jax.experimental.pallas import tpu as pltpu
from jax.experimental.pallas import tpu_sc as plsc

_B = 4
_N = 8192
_LD = 1024
_SCALES = ((1024, 13), (512, 26), (256, 52))
_NC, _NS, _NW = 2, 16, 32
_CPW = _LD // _NW
_K = 4


def _sc_project(pnf, flat, HWp):
    mesh = plsc.VectorSubcoreMesh(core_axis_name="c", subcore_axis_name="s")
    ngroups = _CPW // _K

    @functools.partial(
        pl.kernel, mesh=mesh,
        compiler_params=pltpu.CompilerParams(needs_layout_passes=False),
        out_type=(jax.ShapeDtypeStruct((_B, _LD, HWp), jnp.float32),
                  jax.ShapeDtypeStruct((_B, HWp), jnp.float32)),
        scratch_types=(
            [pltpu.VMEM((_N,), jnp.int32)]
            + [pltpu.VMEM((_N,), jnp.float32) for _ in range(2 * _K)]
            + [pltpu.VMEM((HWp,), jnp.float32) for _ in range(2 * _K + 1)]
            + [pltpu.SemaphoreType.DMA, pltpu.SemaphoreType.DMA]),
    )
    def proj(pnf_hbm, flat_hbm, g_hbm, cnt_hbm, idx_v, *rest):
        feats = rest[:2 * _K]
        accs = rest[2 * _K:4 * _K]
        cacc = rest[4 * _K]
        sem, wsem = rest[-2], rest[-1]
        wid = lax.axis_index("s") * _NC + lax.axis_index("c")
        c0 = wid * _CPW
        ones16 = jnp.ones((16,), jnp.float32)
        zeros16 = jnp.zeros((16,), jnp.float32)

        seq = [(b, g0) for b in range(_B) for g0 in range(ngroups)]

        def fire(step):
            b, g0 = seq[step]
            s = (step % 2) * _K
            return [pltpu.async_copy(pnf_hbm.at[b, c0 + g0 * _K + r],
                                     feats[s + r], sem) for r in range(_K)]

        loads = {0: fire(0)}
        wbacks = {}
        for step, (b, g0) in enumerate(seq):
            s = (step % 2) * _K
            if g0 == 0:
                pltpu.sync_copy(flat_hbm.at[b], idx_v)

                @pl.when(wid == b)
                def _counts():
                    @plsc.parallel_loop(0, HWp // 16, unroll=4)
                    def _z(i):
                        cacc[pl.ds(i * 16, 16)] = zeros16

                    def cb(j, carry):
                        plsc.addupdate_scatter(
                            cacc, [idx_v[pl.ds(j * 16, 16)]], ones16)
                        return carry
                    lax.fori_loop(0, _N // 16, cb, 0)
                    pltpu.sync_copy(cacc, cnt_hbm.at[b])

            for cp in loads.pop(step):
                cp.wait()
            if step + 1 < len(seq):
                loads[step + 1] = fire(step + 1)

            for cp in wbacks.pop(step - 2, ()):
                cp.wait()

            @plsc.parallel_loop(0, HWp // 16, unroll=4)
            def _zero(i):
                for r in range(_K):
                    accs[s + r][pl.ds(i * 16, 16)] = zeros16

            @plsc.parallel_loop(0, _N // 16, unroll=4)
            def _scatter(j):
                idx = idx_v[pl.ds(j * 16, 16)]
                for r in range(_K):
                    plsc.addupdate_scatter(
                        accs[s + r], [idx], feats[s + r][pl.ds(j * 16, 16)])

            wbacks[step] = [
                pltpu.async_copy(accs[s + r], g_hbm.at[b, c0 + g0 * _K + r],
                                 wsem) for r in range(_K)]
        for cps in wbacks.values():
            for cp in cps:
                cp.wait()

    return proj(pnf, flat)


def _ln_cols(x, g, b):
    m = jnp.mean(x, axis=0, keepdims=True)
    v = jnp.mean((x - m) * (x - m), axis=0, keepdims=True)
    return (x - m) * jax.lax.rsqrt(v + 1e-5) * g + b


def _qkv_body(yolo_ref, g_ref, cnt_ref, yr_w, yr_b, lr_w, lr_b, q_w, q_b,
              k_w, k_b, v_w, v_b, n1_g, n1_b,
              q_out, k_out, v_out, *, HW):
    yolo = yolo_ref[0]
    g = g_ref[0, :, :HW]
    cnt = cnt_ref[0, :, :HW]
    lid = jnp.maximum(g / (cnt + 1e-6), 0.0)
    lid = jnp.where(lid == 0.0, 1e-5, lid)
    lidar = jnp.log1p(lid)

    bf = jnp.bfloat16
    yf = _ln_cols(jnp.dot(yr_w[...].astype(bf), yolo.astype(bf),
                          preferred_element_type=jnp.float32) + yr_b[...],
                  n1_g[...], n1_b[...])
    lf = _ln_cols(jnp.dot(lr_w[...].astype(bf), lidar.astype(bf),
                          preferred_element_type=jnp.float32) + lr_b[...],
                  n1_g[...], n1_b[...])

    yf16, lf16 = yf.astype(bf), lf.astype(bf)
    q_out[0] = jnp.dot(q_w[...].astype(bf), yf16,
                       preferred_element_type=jnp.float32) + q_b[...]
    k_out[0] = jnp.dot(k_w[...].astype(bf), lf16,
                       preferred_element_type=jnp.float32) + k_b[...]
    v_out[0] = jnp.dot(v_w[...].astype(bf), lf16,
                       preferred_element_type=jnp.float32) + v_b[...]


def _attn_body(yolo_ref, q_ref, k_ref, v_ref, o_w, o_b, n2_g, n2_b,
               out_ref, *, Ch):
    bf = jnp.bfloat16
    q, k, v = q_ref[0].astype(bf), k_ref[0].astype(bf), v_ref[0]

    tn = (((0,), (0,)), ((), ()))
    scores = jax.lax.dot_general(
        q, k, tn, preferred_element_type=jnp.float32) / np.sqrt(Ch)
    scores -= jnp.max(scores, axis=-1, keepdims=True)
    e = jnp.exp(scores)
    attn = e / jnp.sum(e, axis=-1, keepdims=True)

    nt = (((1,), (1,)), ((), ()))
    fus = jax.lax.dot_general(v.astype(bf), attn.astype(bf), nt,
                              preferred_element_type=jnp.float32)

    o = jnp.dot(o_w[...].astype(bf), fus.astype(bf),
                preferred_element_type=jnp.float32) + o_b[...]
    out_ref[0] = yolo_ref[0] + 0.5 * _ln_cols(o, n2_g[...], n2_b[...])


def _col(x):
    return x.reshape(-1, 1)


def _fusion(yolo, g_sums, cnt, p, C, HW, HWp):
    Ch = C // 2

    w1 = [p['yr_w'], _col(p['yr_b']), p['lr_w'], _col(p['lr_b']),
          p['q_w'], _col(p['q_b']), p['k_w'], _col(p['k_b']),
          p['v_w'], _col(p['v_b']), _col(p['n1_g']), _col(p['n1_b'])]
    w1_specs = [pl.BlockSpec(w.shape, lambda b: (0, 0)) for w in w1]
    qkv_shape = jax.ShapeDtypeStruct((_B, Ch, HW), jnp.float32)
    qkv_spec = pl.BlockSpec((1, Ch, HW), lambda b: (b, 0, 0))
    q, k, v = pl.pallas_call(
        functools.partial(_qkv_body, HW=HW),
        grid=(_B,),
        in_specs=[
            pl.BlockSpec((1, C, HW), lambda b: (b, 0, 0)),
            pl.BlockSpec((1, _LD, HWp), lambda b: (b, 0, 0)),
            pl.BlockSpec((1, 1, HWp), lambda b: (b, 0, 0)),
        ] + w1_specs,
        out_specs=(qkv_spec, qkv_spec, qkv_spec),
        out_shape=(qkv_shape, qkv_shape, qkv_shape),
    )(yolo, g_sums, cnt.reshape(_B, 1, HWp), *w1)

    w2 = [p['o_w'], _col(p['o_b']), _col(p['n2_g']), _col(p['n2_b'])]
    w2_specs = [pl.BlockSpec(w.shape, lambda b: (0, 0)) for w in w2]
    return pl.pallas_call(
        functools.partial(_attn_body, Ch=Ch),
        grid=(_B,),
        in_specs=[pl.BlockSpec((1, C, HW), lambda b: (b, 0, 0)),
                  qkv_spec, qkv_spec, qkv_spec] + w2_specs,
        out_specs=pl.BlockSpec((1, C, HW), lambda b: (b, 0, 0)),
        out_shape=jax.ShapeDtypeStruct((_B, C, HW), jnp.float32),
    )(yolo, q, k, v, *w2)


def kernel(point_net_features, yolo13, yolo26, yolo52,
           coords13, coords26, coords52, params):
    yolos = (yolo13, yolo26, yolo52)
    coords = (coords13, coords26, coords52)
    projs = []
    for i, (C, H) in enumerate(_SCALES):
        HWp = -(-(H * H) // 16) * 16
        flat = (coords[i][:, :, 0] * H + coords[i][:, :, 1]).astype(jnp.int32)
        projs.append(_sc_project(point_net_features, flat, HWp))
    outs = []
    for i, (C, H) in enumerate(_SCALES):
        HW = H * H
        HWp = -(-HW // 16) * 16
        g_sums, cnt = projs[i]
        out = _fusion(yolos[i].reshape(_B, C, HW), g_sums, cnt,
                      params[i], C, HW, HWp)
        outs.append(out.reshape(_B, C, H, H))
    return tuple(outs)


# speedup vs baseline: 1.2192x; 1.2192x over previous
import functools

import jax
import jax.numpy as jnp
import numpy as np
from jax import lax
from jax.experimental import pallas as pl
from jax.experimental.pallas import tpu as pltpu
from jax.experimental.pallas import tpu_sc as plsc

_B = 4
_N = 8192
_LD = 1024
_SCALES = ((1024, 13), (512, 26), (256, 52))
_NC, _NS, _NW = 2, 16, 32
_CPW = _LD // _NW
_K = 4


def _sc_project(pnf, flat, HWp):
    mesh = plsc.VectorSubcoreMesh(core_axis_name="c", subcore_axis_name="s")
    ngroups = _CPW // _K

    @functools.partial(
        pl.kernel, mesh=mesh,
        compiler_params=pltpu.CompilerParams(needs_layout_passes=False),
        out_type=(jax.ShapeDtypeStruct((_B, _LD, HWp), jnp.float32),
                  jax.ShapeDtypeStruct((_B, 1, HWp), jnp.float32)),
        scratch_types=(
            [pltpu.VMEM((_N,), jnp.int32)]
            + [pltpu.VMEM((_N,), jnp.float32) for _ in range(2 * _K)]
            + [pltpu.VMEM((HWp,), jnp.float32) for _ in range(2 * _K + 1)]
            + [pltpu.SemaphoreType.DMA, pltpu.SemaphoreType.DMA]),
    )
    def proj(pnf_hbm, flat_hbm, g_hbm, cnt_hbm, idx_v, *rest):
        feats = rest[:2 * _K]
        accs = rest[2 * _K:4 * _K]
        cacc = rest[4 * _K]
        sem, wsem = rest[-2], rest[-1]
        wid = lax.axis_index("s") * _NC + lax.axis_index("c")
        c0 = wid * _CPW
        ones16 = jnp.ones((16,), jnp.float32)
        zeros16 = jnp.zeros((16,), jnp.float32)

        seq = [(b, g0) for b in range(_B) for g0 in range(ngroups)]

        def fire(step):
            b, g0 = seq[step]
            s = (step % 2) * _K
            return [pltpu.async_copy(pnf_hbm.at[b, c0 + g0 * _K + r],
                                     feats[s + r], sem) for r in range(_K)]

        loads = {0: fire(0)}
        wbacks = {}
        for step, (b, g0) in enumerate(seq):
            s = (step % 2) * _K
            if g0 == 0:
                pltpu.sync_copy(flat_hbm.at[b], idx_v)

                @pl.when(wid == b)
                def _counts():
                    @plsc.parallel_loop(0, HWp // 16, unroll=4)
                    def _z(i):
                        cacc[pl.ds(i * 16, 16)] = zeros16

                    def cb(j, carry):
                        plsc.addupdate_scatter(
                            cacc, [idx_v[pl.ds(j * 16, 16)]], ones16)
                        return carry
                    lax.fori_loop(0, _N // 16, cb, 0)
                    pltpu.sync_copy(cacc, cnt_hbm.at[b, 0])

            for cp in loads.pop(step):
                cp.wait()
            if step + 1 < len(seq):
                loads[step + 1] = fire(step + 1)

            for cp in wbacks.pop(step - 2, ()):
                cp.wait()

            @plsc.parallel_loop(0, HWp // 16, unroll=4)
            def _zero(i):
                for r in range(_K):
                    accs[s + r][pl.ds(i * 16, 16)] = zeros16

            @plsc.parallel_loop(0, _N // 16, unroll=4)
            def _scatter(j):
                idx = idx_v[pl.ds(j * 16, 16)]
                for r in range(_K):
                    plsc.addupdate_scatter(
                        accs[s + r], [idx], feats[s + r][pl.ds(j * 16, 16)])

            wbacks[step] = [
                pltpu.async_copy(accs[s + r], g_hbm.at[b, c0 + g0 * _K + r],
                                 wsem) for r in range(_K)]
        for cps in wbacks.values():
            for cp in cps:
                cp.wait()

    return proj(pnf, flat)


_NB = 512


def _tc_proj_body(idx_ref, f_ref, out_ref, cnt_ref, *, HWp, nsteps):
    n = pl.program_id(1)

    @pl.when(n == 0)
    def _init():
        out_ref[...] = jnp.zeros_like(out_ref)
        cnt_ref[...] = jnp.zeros_like(cnt_ref)

    flat = idx_ref[0, :, pl.ds(n * _NB, _NB)]
    cell = jax.lax.broadcasted_iota(jnp.int32, (HWp, _NB), 0)
    oh = (cell == flat).astype(jnp.bfloat16)

    f_hi = f_ref[0].astype(jnp.bfloat16)
    nt = (((1,), (1,)), ((), ()))
    out_ref[0] += jax.lax.dot_general(
        f_hi, oh, nt, preferred_element_type=jnp.float32)
    ones = jnp.ones((8, _NB), jnp.bfloat16)
    cnt_ref[0] += jax.lax.dot_general(
        ones, oh, nt, preferred_element_type=jnp.float32)[0:1]


def _tc_project(pnf, flat, HWp):
    nsteps = _N // _NB
    body = functools.partial(_tc_proj_body, HWp=HWp, nsteps=nsteps)
    return pl.pallas_call(
        body,
        grid=(_B, nsteps),
        in_specs=[
            pl.BlockSpec((1, 1, _N), lambda b, n: (b, 0, 0)),
            pl.BlockSpec((1, _LD, _NB), lambda b, n: (b, 0, n)),
        ],
        out_specs=(pl.BlockSpec((1, _LD, HWp), lambda b, n: (b, 0, 0)),
                   pl.BlockSpec((1, 1, HWp), lambda b, n: (b, 0, 0))),
        out_shape=(jax.ShapeDtypeStruct((_B, _LD, HWp), jnp.float32),
                   jax.ShapeDtypeStruct((_B, 1, HWp), jnp.float32)),
    )(flat.reshape(_B, 1, _N), pnf)


def _ln_cols(x, g, b):
    m = jnp.mean(x, axis=0, keepdims=True)
    v = jnp.mean((x - m) * (x - m), axis=0, keepdims=True)
    return (x - m) * jax.lax.rsqrt(v + 1e-5) * g + b


def _qkv_body(yolo_ref, g_ref, cnt_ref, yr_w, yr_b, lr_w, lr_b, q_w, q_b,
              k_w, k_b, v_w, v_b, n1_g, n1_b,
              q_out, k_out, v_out, *, HW):
    yolo = yolo_ref[0]
    g = g_ref[0, :, :HW]
    cnt = cnt_ref[0, :, :HW]
    lid = jnp.maximum(g / (cnt + 1e-6), 0.0)
    lid = jnp.where(lid == 0.0, 1e-5, lid)
    lidar = jnp.log1p(lid)

    bf = jnp.bfloat16
    yf = _ln_cols(jnp.dot(yr_w[...].astype(bf), yolo.astype(bf),
                          preferred_element_type=jnp.float32) + yr_b[...],
                  n1_g[...], n1_b[...])
    lf = _ln_cols(jnp.dot(lr_w[...].astype(bf), lidar.astype(bf),
                          preferred_element_type=jnp.float32) + lr_b[...],
                  n1_g[...], n1_b[...])

    yf16, lf16 = yf.astype(bf), lf.astype(bf)
    q_out[0] = jnp.dot(q_w[...].astype(bf), yf16,
                       preferred_element_type=jnp.float32) + q_b[...]
    k_out[0] = jnp.dot(k_w[...].astype(bf), lf16,
                       preferred_element_type=jnp.float32) + k_b[...]
    v_out[0] = jnp.dot(v_w[...].astype(bf), lf16,
                       preferred_element_type=jnp.float32) + v_b[...]


def _attn_body(yolo_ref, q_ref, k_ref, v_ref, o_w, o_b, n2_g, n2_b,
               out_ref, *, Ch):
    bf = jnp.bfloat16
    q, k, v = q_ref[0].astype(bf), k_ref[0].astype(bf), v_ref[0]

    tn = (((0,), (0,)), ((), ()))
    scores = jax.lax.dot_general(
        q, k, tn, preferred_element_type=jnp.float32) / np.sqrt(Ch)
    scores -= jnp.max(scores, axis=-1, keepdims=True)
    e = jnp.exp(scores)
    attn = e / jnp.sum(e, axis=-1, keepdims=True)

    nt = (((1,), (1,)), ((), ()))
    fus = jax.lax.dot_general(v.astype(bf), attn.astype(bf), nt,
                              preferred_element_type=jnp.float32)

    o = jnp.dot(o_w[...].astype(bf), fus.astype(bf),
                preferred_element_type=jnp.float32) + o_b[...]
    out_ref[0] = yolo_ref[0] + 0.5 * _ln_cols(o, n2_g[...], n2_b[...])


def _col(x):
    return x.reshape(-1, 1)


def _fusion(yolo, g_sums, cnt, p, C, HW, HWp):
    Ch = C // 2

    w1 = [p['yr_w'], _col(p['yr_b']), p['lr_w'], _col(p['lr_b']),
          p['q_w'], _col(p['q_b']), p['k_w'], _col(p['k_b']),
          p['v_w'], _col(p['v_b']), _col(p['n1_g']), _col(p['n1_b'])]
    w1_specs = [pl.BlockSpec(w.shape, lambda b: (0, 0)) for w in w1]
    qkv_shape = jax.ShapeDtypeStruct((_B, Ch, HW), jnp.float32)
    qkv_spec = pl.BlockSpec((1, Ch, HW), lambda b: (b, 0, 0))
    q, k, v = pl.pallas_call(
        functools.partial(_qkv_body, HW=HW),
        grid=(_B,),
        in_specs=[
            pl.BlockSpec((1, C, HW), lambda b: (b, 0, 0)),
            pl.BlockSpec((1, _LD, HWp), lambda b: (b, 0, 0)),
            pl.BlockSpec((1, 1, HWp), lambda b: (b, 0, 0)),
        ] + w1_specs,
        out_specs=(qkv_spec, qkv_spec, qkv_spec),
        out_shape=(qkv_shape, qkv_shape, qkv_shape),
    )(yolo, g_sums, cnt, *w1)

    w2 = [p['o_w'], _col(p['o_b']), _col(p['n2_g']), _col(p['n2_b'])]
    w2_specs = [pl.BlockSpec(w.shape, lambda b: (0, 0)) for w in w2]
    return pl.pallas_call(
        functools.partial(_attn_body, Ch=Ch),
        grid=(_B,),
        in_specs=[pl.BlockSpec((1, C, HW), lambda b: (b, 0, 0)),
                  qkv_spec, qkv_spec, qkv_spec] + w2_specs,
        out_specs=pl.BlockSpec((1, C, HW), lambda b: (b, 0, 0)),
        out_shape=jax.ShapeDtypeStruct((_B, C, HW), jnp.float32),
    )(yolo, q, k, v, *w2)


def kernel(point_net_features, yolo13, yolo26, yolo52,
           coords13, coords26, coords52, params):
    yolos = (yolo13, yolo26, yolo52)
    coords = (coords13, coords26, coords52)
    flats = [
        (coords[i][:, :, 0] * H + coords[i][:, :, 1]).astype(jnp.int32)
        for i, (C, H) in enumerate(_SCALES)]
    projs = [None, None, None]
    projs[2] = _sc_project(point_net_features, flats[2], 2704)
    projs[0] = _tc_project(point_net_features, flats[0], 176)
    projs[1] = _tc_project(point_net_features, flats[1], 688)
    outs = []
    for i, (C, H) in enumerate(_SCALES):
        HW = H * H
        HWp = -(-HW // 16) * 16
        g_sums, cnt = projs[i]
        out = _fusion(yolos[i].reshape(_B, C, HW), g_sums, cnt,
                      params[i], C, HW, HWp)
        outs.append(out.reshape(_B, C, H, H))
    return tuple(outs)
